# baseline (speedup 1.0000x reference)
import jax
import jax.numpy as jnp
from jax import lax
from jax.experimental import pallas as pl
from jax.experimental.pallas import tpu as pltpu
from jax.experimental.pallas import tpu_sc as plsc

_N = 100000
_E = 3200000
_NC = 2
_NS = 16
_NW = _NC * _NS
_L = 128
_ROWS = _E // _L
_RC = 16
_NB = 8
_NCHUNK = _ROWS // _RC
_CW = _NCHUNK // _NW
_CREM = _NCHUNK - _CW * _NW
_TAIL0 = _NCHUNK * _RC


def _pre_body(x0, x1, x2, Wl, bl, Wn, We, bc, g0, g1, g2, s0, s1, s2):
    gs = (g0, g1, g2)
    ss = (s0, s1, s2)
    for t, xr in enumerate((x0, x1, x2)):
        x = xr[...]
        Wg = jnp.dot(Wl[t], We[t], preferred_element_type=jnp.float32)
        Ws = jnp.dot(Wl[t], Wn[t], preferred_element_type=jnp.float32)
        bg = jnp.dot(bl[t], We[t], preferred_element_type=jnp.float32)
        bs = (jnp.dot(bl[t], Wn[t],
                      preferred_element_type=jnp.float32) + bc[t])
        g = Wg[0][:, None] * x[0:1, :]
        s = Ws[0][:, None] * x[0:1, :]
        for k in range(1, 4):
            g = g + Wg[k][:, None] * x[k:k + 1, :]
            s = s + Ws[k][:, None] * x[k:k + 1, :]
        gs[t][...] = g + bg[:, None]
        ss[t][...] = s + bs[:, None]


def _post_body(s0, s1, s2, P, y0, y1, y2):
    for t, (sb, yb) in enumerate(((s0, y0), (s1, y1), (s2, y2))):
        aggs = []
        for c in range(2):
            r0 = (t * 2 + c) * _NW
            aggs.append(jnp.sum(P[r0:r0 + _NW, :], axis=0, keepdims=True))
        z = sb[...] + jnp.concatenate(aggs, axis=0)
        yb[...] = jax.nn.sigmoid(jnp.maximum(z, 0.0))


def _sc_edge_kernel(edges, g0, g1, g2, outP, src_v, dst_v, val_v, acc, sem):
    cid = lax.axis_index("c")
    sid = lax.axis_index("s")
    wid = cid * _NS + sid

    z16 = jnp.zeros((16,), jnp.float32)

    def zero_acc():
        def zbody(i, _):
            for s in range(8):
                acc[pl.ds(i * 128 + s * 16, 16)] = z16
            return 0
        lax.fori_loop(0, _N // 128, zbody, 0)
        for s in range(_N // 128 * 128, _N, 16):
            acc[pl.ds(s, 16)] = z16

    def do_chunk(t, g, ch, row0, nrows_static):
        pltpu.sync_copy(
            edges.at[pl.ds(2 * t * _ROWS + row0, nrows_static)],
            src_v.at[pl.ds(0, nrows_static)])
        pltpu.sync_copy(
            edges.at[pl.ds((2 * t + 1) * _ROWS + row0, nrows_static)],
            dst_v.at[pl.ds(0, nrows_static)])

        def conv_row(row, _):
            def conv_v(v, _):
                sl = pl.ds(v * 16, 16)
                src_v[row, sl] = src_v[row, sl] * 2 + ch
                return 0
            lax.fori_loop(0, _L // 16, conv_v, 0)
            return 0
        lax.fori_loop(0, nrows_static, conv_row, 0)

        for r in range(nrows_static // _NB):
            cps = [pltpu.async_copy(g.at[src_v.at[r * _NB + b]],
                                    val_v.at[b], sem)
                   for b in range(_NB)]
            for b in range(_NB):
                cps[b].wait()
                row = r * _NB + b

                def scat_v(v, _, row=row, b=b):
                    dst16 = dst_v[row, pl.ds(v * 16, 16)]
                    val16 = val_v[b, pl.ds(v * 16, 16)]
                    plsc.addupdate_scatter(acc, [dst16], val16)
                    return 0
                lax.fori_loop(0, _L // 16, scat_v, 0)

    for t, g in enumerate((g0, g1, g2)):
        def ch_body(ch, _, t=t, g=g):
            zero_acc()

            def chunk_body(k, _):
                do_chunk(t, g, ch, (wid + k * _NW) * _RC, _RC)
                return 0
            lax.fori_loop(0, _CW, chunk_body, 0)

            @pl.when(wid < _CREM)
            def _extra():
                do_chunk(t, g, ch, (wid + _CW * _NW) * _RC, _RC)

            @pl.when(wid == _NW - 1)
            def _tail():
                do_chunk(t, g, ch, _TAIL0, _ROWS - _NCHUNK * _RC)

            pltpu.sync_copy(acc, outP.at[(t * 2 + ch) * _NW + wid])
            return 0
        lax.fori_loop(0, 2, ch_body, 0)


def _make_sc_call(n):
    mesh = plsc.VectorSubcoreMesh(core_axis_name="c", subcore_axis_name="s")
    return pl.kernel(
        _sc_edge_kernel,
        out_type=jax.ShapeDtypeStruct((6 * _NW, n), jnp.float32),
        mesh=mesh,
        scratch_types=[
            pltpu.VMEM((_RC, _L), jnp.int32),
            pltpu.VMEM((_RC, _L), jnp.int32),
            pltpu.VMEM((_NB, _L), jnp.float32),
            pltpu.VMEM((n,), jnp.float32),
            pltpu.SemaphoreType.DMA,
        ],
        compiler_params=pltpu.CompilerParams(
            use_tc_tiling_on_sc=False, needs_layout_passes=False),
    )


def kernel(x0, x1, x2, edges, W_lin, b_lin, Wn, We, bc):
    n = x0.shape[0]

    gT0, gT1, gT2, sT0, sT1, sT2 = pl.pallas_call(
        _pre_body,
        out_shape=[jax.ShapeDtypeStruct((2, n), jnp.float32)] * 6,
    )(x0.T, x1.T, x2.T, W_lin, b_lin, Wn, We, bc)

    P = _make_sc_call(n)(edges.reshape(-1, _L),
                         gT0.T.reshape(-1), gT1.T.reshape(-1),
                         gT2.T.reshape(-1))

    bn = 12800
    grid = (n + bn - 1) // bn
    yT0, yT1, yT2 = pl.pallas_call(
        _post_body,
        grid=(grid,),
        in_specs=[pl.BlockSpec((2, bn), lambda i: (0, i))] * 3
        + [pl.BlockSpec((6 * _NW, bn), lambda i: (0, i))],
        out_specs=[pl.BlockSpec((2, bn), lambda i: (0, i))] * 3,
        out_shape=[jax.ShapeDtypeStruct((2, n), jnp.float32)] * 3,
    )(sT0, sT1, sT2, P)

    return (yT0.T, yT1.T, yT2.T)


# speedup vs baseline: 1.0837x; 1.0837x over previous
import jax
import jax.numpy as jnp
from jax import lax
from jax.experimental import pallas as pl
from jax.experimental.pallas import tpu as pltpu
from jax.experimental.pallas import tpu_sc as plsc

_N = 100000
_E = 3200000
_NC = 2
_NS = 16
_NW = _NC * _NS
_L = 128
_ROWS = _E // _L
_RC = 16
_NB = 8
_NCHUNK = _ROWS // _RC
_CW = _NCHUNK // _NW
_CREM = _NCHUNK - _CW * _NW
_TAIL0 = _NCHUNK * _RC


def _pre_body(x0, x1, x2, Wl, bl, Wn, We, bc, g0, g1, g2, s0, s1, s2):
    gs = (g0, g1, g2)
    ss = (s0, s1, s2)
    for t, xr in enumerate((x0, x1, x2)):
        x = xr[...]
        Wg = jnp.dot(Wl[t], We[t], preferred_element_type=jnp.float32)
        Ws = jnp.dot(Wl[t], Wn[t], preferred_element_type=jnp.float32)
        bg = jnp.dot(bl[t], We[t], preferred_element_type=jnp.float32)
        bs = (jnp.dot(bl[t], Wn[t],
                      preferred_element_type=jnp.float32) + bc[t])
        g = Wg[0][:, None] * x[0:1, :]
        s = Ws[0][:, None] * x[0:1, :]
        for k in range(1, 4):
            g = g + Wg[k][:, None] * x[k:k + 1, :]
            s = s + Ws[k][:, None] * x[k:k + 1, :]
        gs[t][...] = g + bg[:, None]
        ss[t][...] = s + bs[:, None]


def _post_body(s0, s1, s2, P, y0, y1, y2):
    for t, (sb, yb) in enumerate(((s0, y0), (s1, y1), (s2, y2))):
        aggs = []
        for c in range(2):
            r0 = (t * 2 + c) * _NW
            aggs.append(jnp.sum(P[r0:r0 + _NW, :], axis=0, keepdims=True))
        z = sb[...] + jnp.concatenate(aggs, axis=0)
        yb[...] = jax.nn.sigmoid(jnp.maximum(z, 0.0))


def _sc_edge_kernel(edges, gflat, outP, src_v, dst_v, val_v, acc, sem):
    cid = lax.axis_index("c")
    sid = lax.axis_index("s")
    wid = cid * _NS + sid

    z16 = jnp.zeros((16,), jnp.float32)

    def zero_acc():
        def zbody(i, _):
            for s in range(8):
                acc[pl.ds(i * 128 + s * 16, 16)] = z16
            return 0
        lax.fori_loop(0, _N // 128, zbody, 0)
        for s in range(_N // 128 * 128, _N, 16):
            acc[pl.ds(s, 16)] = z16

    def do_chunk(p, row0, nrows_static):
        srow0 = (p // 2) * 2 * _ROWS + row0
        drow0 = ((p // 2) * 2 + 1) * _ROWS + row0
        pltpu.sync_copy(edges.at[pl.ds(srow0, nrows_static)],
                        src_v.at[pl.ds(0, nrows_static)])
        pltpu.sync_copy(edges.at[pl.ds(drow0, nrows_static)],
                        dst_v.at[pl.ds(0, nrows_static)])
        slab = gflat.at[pl.ds(p * _N, _N)]

        for r in range(nrows_static // _NB):
            cps = [pltpu.async_copy(slab.at[src_v.at[r * _NB + b]],
                                    val_v.at[b], sem)
                   for b in range(_NB)]
            for b in range(_NB):
                cps[b].wait()
                row = r * _NB + b
                for v in range(_L // 16):
                    dst16 = dst_v[row, pl.ds(v * 16, 16)]
                    val16 = val_v[b, pl.ds(v * 16, 16)]
                    plsc.addupdate_scatter(acc, [dst16], val16)

    def p_body(p, _):
        zero_acc()

        def chunk_body(k, _):
            do_chunk(p, (wid + k * _NW) * _RC, _RC)
            return 0
        lax.fori_loop(0, _CW, chunk_body, 0)

        @pl.when(wid < _CREM)
        def _extra():
            do_chunk(p, (wid + _CW * _NW) * _RC, _RC)

        @pl.when(wid == _NW - 1)
        def _tail():
            do_chunk(p, _TAIL0, _ROWS - _NCHUNK * _RC)

        pltpu.sync_copy(acc, outP.at[p * _NW + wid])
        return 0
    lax.fori_loop(0, 6, p_body, 0)


def _make_sc_call(n):
    mesh = plsc.VectorSubcoreMesh(core_axis_name="c", subcore_axis_name="s")
    return pl.kernel(
        _sc_edge_kernel,
        out_type=jax.ShapeDtypeStruct((6 * _NW, n), jnp.float32),
        mesh=mesh,
        scratch_types=[
            pltpu.VMEM((_RC, _L), jnp.int32),
            pltpu.VMEM((_RC, _L), jnp.int32),
            pltpu.VMEM((_NB, _L), jnp.float32),
            pltpu.VMEM((n,), jnp.float32),
            pltpu.SemaphoreType.DMA,
        ],
        compiler_params=pltpu.CompilerParams(
            use_tc_tiling_on_sc=False, needs_layout_passes=False),
    )


def kernel(x0, x1, x2, edges, W_lin, b_lin, Wn, We, bc):
    n = x0.shape[0]

    gT0, gT1, gT2, sT0, sT1, sT2 = pl.pallas_call(
        _pre_body,
        out_shape=[jax.ShapeDtypeStruct((2, n), jnp.float32)] * 6,
    )(x0.T, x1.T, x2.T, W_lin, b_lin, Wn, We, bc)

    gflat = jnp.concatenate(
        [gT0.reshape(-1), gT1.reshape(-1), gT2.reshape(-1)])
    P = _make_sc_call(n)(edges.reshape(-1, _L), gflat)

    bn = 12800
    grid = (n + bn - 1) // bn
    yT0, yT1, yT2 = pl.pallas_call(
        _post_body,
        grid=(grid,),
        in_specs=[pl.BlockSpec((2, bn), lambda i: (0, i))] * 3
        + [pl.BlockSpec((6 * _NW, bn), lambda i: (0, i))],
        out_specs=[pl.BlockSpec((2, bn), lambda i: (0, i))] * 3,
        out_shape=[jax.ShapeDtypeStruct((2, n), jnp.float32)] * 3,
    )(sT0, sT1, sT2, P)

    return (yT0.T, yT1.T, yT2.T)


# speedup vs baseline: 1.3939x; 1.2863x over previous
import jax
import jax.numpy as jnp
from jax import lax
from jax.experimental import pallas as pl
from jax.experimental.pallas import tpu as pltpu
from jax.experimental.pallas import tpu_sc as plsc

_N = 100000
_E = 3200000
_NC = 2
_NS = 16
_NW = _NC * _NS
_L = 128
_ROWS = _E // _L
_RC = 32
_NB = 16
_NCHUNK = _ROWS // _RC
_CW = _NCHUNK // _NW
_CREM = _NCHUNK - _CW * _NW
_TAIL0 = _NCHUNK * _RC


def _pre_body(x0, x1, x2, Wl, bl, Wn, We, bc, g0, g1, g2, s0, s1, s2):
    gs = (g0, g1, g2)
    ss = (s0, s1, s2)
    for t, xr in enumerate((x0, x1, x2)):
        x = xr[...]
        Wg = jnp.dot(Wl[t], We[t], preferred_element_type=jnp.float32)
        Ws = jnp.dot(Wl[t], Wn[t], preferred_element_type=jnp.float32)
        bg = jnp.dot(bl[t], We[t], preferred_element_type=jnp.float32)
        bs = (jnp.dot(bl[t], Wn[t],
                      preferred_element_type=jnp.float32) + bc[t])
        g = Wg[0][:, None] * x[0:1, :]
        s = Ws[0][:, None] * x[0:1, :]
        for k in range(1, 4):
            g = g + Wg[k][:, None] * x[k:k + 1, :]
            s = s + Ws[k][:, None] * x[k:k + 1, :]
        gs[t][...] = g + bg[:, None]
        ss[t][...] = s + bs[:, None]


def _post_body(s0, s1, s2, P, y0, y1, y2):
    for t, (sb, yb) in enumerate(((s0, y0), (s1, y1), (s2, y2))):
        aggs = []
        for c in range(2):
            r0 = (t * 2 + c) * _NW
            aggs.append(jnp.sum(P[r0:r0 + _NW, :], axis=0, keepdims=True))
        z = sb[...] + jnp.concatenate(aggs, axis=0)
        yb[...] = jax.nn.sigmoid(jnp.maximum(z, 0.0))


def _sc_edge_kernel(edges, gflat, outP, sd_v, val_v, acc, sem):
    cid = lax.axis_index("c")
    sid = lax.axis_index("s")
    wid = cid * _NS + sid

    z16 = jnp.zeros((16,), jnp.float32)

    def zero_acc():
        def zbody(i, _):
            for s in range(8):
                acc[pl.ds(i * 128 + s * 16, 16)] = z16
            return 0
        lax.fori_loop(0, _N // 128, zbody, 0)
        for s in range(_N // 128 * 128, _N, 16):
            acc[pl.ds(s, 16)] = z16

    def do_chunk(p, row0, nrows_static):
        pltpu.sync_copy(
            edges.at[pl.ds(p // 2, 1), pl.ds(0, 2),
                     pl.ds(row0, nrows_static)],
            sd_v.at[pl.ds(0, 1), pl.ds(0, 2), pl.ds(0, nrows_static)])
        slab = gflat.at[pl.ds(p * _N, _N)]

        for r0 in range(0, nrows_static, _NB):
            nb = min(_NB, nrows_static - r0)
            cps = [pltpu.async_copy(slab.at[sd_v.at[0, 0, r0 + b]],
                                    val_v.at[b], sem)
                   for b in range(nb)]
            for b in range(nb):
                cps[b].wait()
                row = r0 + b
                for v in range(_L // 16):
                    dst16 = sd_v[0, 1, row, pl.ds(v * 16, 16)]
                    val16 = val_v[b, pl.ds(v * 16, 16)]
                    plsc.addupdate_scatter(acc, [dst16], val16)

    def p_body(p, _):
        zero_acc()

        def chunk_body(k, _):
            do_chunk(p, (wid + k * _NW) * _RC, _RC)
            return 0
        lax.fori_loop(0, _CW, chunk_body, 0)

        @pl.when(wid < _CREM)
        def _extra():
            do_chunk(p, (wid + _CW * _NW) * _RC, _RC)

        @pl.when(wid == _NW - 1)
        def _tail():
            do_chunk(p, _TAIL0, _ROWS - _NCHUNK * _RC)

        pltpu.sync_copy(acc, outP.at[p * _NW + wid])
        return 0
    lax.fori_loop(0, 6, p_body, 0)


def _make_sc_call(n):
    mesh = plsc.VectorSubcoreMesh(core_axis_name="c", subcore_axis_name="s")
    return pl.kernel(
        _sc_edge_kernel,
        out_type=jax.ShapeDtypeStruct((6 * _NW, n), jnp.float32),
        mesh=mesh,
        scratch_types=[
            pltpu.VMEM((1, 2, _RC, _L), jnp.int32),
            pltpu.VMEM((_NB, _L), jnp.float32),
            pltpu.VMEM((n,), jnp.float32),
            pltpu.SemaphoreType.DMA,
        ],
        compiler_params=pltpu.CompilerParams(
            use_tc_tiling_on_sc=False, needs_layout_passes=False),
    )


def kernel(x0, x1, x2, edges, W_lin, b_lin, Wn, We, bc):
    n = x0.shape[0]

    gT0, gT1, gT2, sT0, sT1, sT2 = pl.pallas_call(
        _pre_body,
        out_shape=[jax.ShapeDtypeStruct((2, n), jnp.float32)] * 6,
    )(x0.T, x1.T, x2.T, W_lin, b_lin, Wn, We, bc)

    gflat = jnp.concatenate(
        [gT0.reshape(-1), gT1.reshape(-1), gT2.reshape(-1)])
    P = _make_sc_call(n)(edges.reshape(3, 2, _ROWS, _L), gflat)

    bn = 12800
    grid = (n + bn - 1) // bn
    yT0, yT1, yT2 = pl.pallas_call(
        _post_body,
        grid=(grid,),
        in_specs=[pl.BlockSpec((2, bn), lambda i: (0, i))] * 3
        + [pl.BlockSpec((6 * _NW, bn), lambda i: (0, i))],
        out_specs=[pl.BlockSpec((2, bn), lambda i: (0, i))] * 3,
        out_shape=[jax.ShapeDtypeStruct((2, n), jnp.float32)] * 3,
    )(sT0, sT1, sT2, P)

    return (yT0.T, yT1.T, yT2.T)


# speedup vs baseline: 1.6206x; 1.1627x over previous
import jax
import jax.numpy as jnp
from jax import lax
from jax.experimental import pallas as pl
from jax.experimental.pallas import tpu as pltpu
from jax.experimental.pallas import tpu_sc as plsc

_N = 100000
_E = 3200000
_NC = 2
_NS = 16
_NW = _NC * _NS
_L = 128
_ROWS = _E // _L
_RC = 32
_NB = 16
_NCHUNK = _ROWS // _RC
_CW = _NCHUNK // _NW
_CREM = _NCHUNK - _CW * _NW
_TAIL0 = _NCHUNK * _RC


def _pre_body(x0, x1, x2, Wl, bl, Wn, We, bc, g0, g1, g2, s0, s1, s2):
    gs = (g0, g1, g2)
    ss = (s0, s1, s2)
    for t, xr in enumerate((x0, x1, x2)):
        x = xr[...]
        Wg = jnp.dot(Wl[t], We[t], preferred_element_type=jnp.float32)
        Ws = jnp.dot(Wl[t], Wn[t], preferred_element_type=jnp.float32)
        bg = jnp.dot(bl[t], We[t], preferred_element_type=jnp.float32)
        bs = (jnp.dot(bl[t], Wn[t],
                      preferred_element_type=jnp.float32) + bc[t])
        g = Wg[0][:, None] * x[0:1, :]
        s = Ws[0][:, None] * x[0:1, :]
        for k in range(1, 4):
            g = g + Wg[k][:, None] * x[k:k + 1, :]
            s = s + Ws[k][:, None] * x[k:k + 1, :]
        gs[t][...] = g + bg[:, None]
        ss[t][...] = s + bs[:, None]


def _post_body(s0, s1, s2, P, y0, y1, y2):
    for t, (sb, yb) in enumerate(((s0, y0), (s1, y1), (s2, y2))):
        aggs = []
        for c in range(2):
            r0 = (t * 2 + c) * _NW
            aggs.append(jnp.sum(P[r0:r0 + _NW, :], axis=0, keepdims=True))
        z = sb[...] + jnp.concatenate(aggs, axis=0)
        yb[...] = jax.nn.sigmoid(jnp.maximum(z, 0.0))


def _sc_edge_kernel(edges, gflat, outP, sd_v, val_v, acc, sem, sem2):
    cid = lax.axis_index("c")
    sid = lax.axis_index("s")
    wid = cid * _NS + sid

    z16 = jnp.zeros((16,), jnp.float32)

    def zero_acc():
        def zbody(i, _):
            for s in range(8):
                acc[pl.ds(i * 128 + s * 16, 16)] = z16
            return 0
        lax.fori_loop(0, _N // 128, zbody, 0)
        for s in range(_N // 128 * 128, _N, 16):
            acc[pl.ds(s, 16)] = z16

    def stage(p, row0, nrows_static, s):
        return pltpu.async_copy(
            edges.at[pl.ds(p // 2, 1), pl.ds(0, 2),
                     pl.ds(row0, nrows_static)],
            sd_v.at[pl.ds(s, 1), pl.ds(0, 2), pl.ds(0, nrows_static)],
            sem2)

    def process(p, s, nrows_static):
        slab = gflat.at[pl.ds(p * _N, _N)]
        cps = [pltpu.async_copy(slab.at[sd_v.at[s, 0, b]],
                                val_v.at[b], sem)
               for b in range(nrows_static)]
        for b in range(nrows_static):
            cps[b].wait()
            for v in range(_L // 16):
                dst16 = sd_v[s, 1, b, pl.ds(v * 16, 16)]
                val16 = val_v[b, pl.ds(v * 16, 16)]
                plsc.addupdate_scatter(acc, [dst16], val16)

    def row0_of(p, k):
        return (wid + k * _NW) * _RC

    def p_body(p, _):
        zero_acc()
        stage(p, row0_of(p, 0), _RC, 0).wait()

        def chunk_body(k2, _):
            cp1 = stage(p, row0_of(p, 2 * k2 + 1), _RC, 1)
            process(p, 0, _RC)
            cp1.wait()
            kc = jnp.minimum(2 * k2 + 2, _CW - 1)
            cp0 = stage(p, row0_of(p, kc), _RC, 0)
            process(p, 1, _RC)
            cp0.wait()
            return 0
        lax.fori_loop(0, _CW // 2, chunk_body, 0)

        @pl.when(wid < _CREM)
        def _extra():
            stage(p, row0_of(p, _CW), _RC, 0).wait()
            process(p, 0, _RC)

        @pl.when(wid == _NW - 1)
        def _tail():
            ntail = _ROWS - _NCHUNK * _RC
            stage(p, _TAIL0, ntail, 0).wait()
            process(p, 0, ntail)

        pltpu.sync_copy(acc, outP.at[p * _NW + wid])
        return 0
    lax.fori_loop(0, 6, p_body, 0)


def _make_sc_call(n):
    mesh = plsc.VectorSubcoreMesh(core_axis_name="c", subcore_axis_name="s")
    return pl.kernel(
        _sc_edge_kernel,
        out_type=jax.ShapeDtypeStruct((6 * _NW, n), jnp.float32),
        mesh=mesh,
        scratch_types=[
            pltpu.VMEM((2, 2, _RC, _L), jnp.int32),
            pltpu.VMEM((_RC, _L), jnp.float32),
            pltpu.VMEM((n,), jnp.float32),
            pltpu.SemaphoreType.DMA,
            pltpu.SemaphoreType.DMA,
        ],
        compiler_params=pltpu.CompilerParams(
            use_tc_tiling_on_sc=False, needs_layout_passes=False),
    )


def kernel(x0, x1, x2, edges, W_lin, b_lin, Wn, We, bc):
    n = x0.shape[0]

    gT0, gT1, gT2, sT0, sT1, sT2 = pl.pallas_call(
        _pre_body,
        out_shape=[jax.ShapeDtypeStruct((2, n), jnp.float32)] * 6,
    )(x0.T, x1.T, x2.T, W_lin, b_lin, Wn, We, bc)

    gflat = jnp.concatenate(
        [gT0.reshape(-1), gT1.reshape(-1), gT2.reshape(-1)])
    P = _make_sc_call(n)(edges.reshape(3, 2, _ROWS, _L), gflat)

    bn = 12800
    grid = (n + bn - 1) // bn
    yT0, yT1, yT2 = pl.pallas_call(
        _post_body,
        grid=(grid,),
        in_specs=[pl.BlockSpec((2, bn), lambda i: (0, i))] * 3
        + [pl.BlockSpec((6 * _NW, bn), lambda i: (0, i))],
        out_specs=[pl.BlockSpec((2, bn), lambda i: (0, i))] * 3,
        out_shape=[jax.ShapeDtypeStruct((2, n), jnp.float32)] * 3,
    )(sT0, sT1, sT2, P)

    return (yT0.T, yT1.T, yT2.T)
